# baseline (speedup 1.0000x reference)
import functools

import jax
import jax.numpy as jnp
from jax.experimental import pallas as pl
from jax.experimental.pallas import tpu as pltpu

_KSIZE = 7
_PAD = _KSIZE // 2
_BN_EPS = 1e-5
_VMEM_LIMIT = 32 << 20


def _ext_pad(W):
    return ((_PAD * W + _PAD) + 127) // 128 * 128


def _gate_kernel(bn_ref, wmat_ref, col_ref, x_ref, s_ref, ext_ref,
                 *, C, H, W, NB):
    HW = H * W
    ep = _ext_pad(W)

    x = x_ref[...]
    mx = jnp.max(x, axis=1)
    mn = jnp.sum(x, axis=1, dtype=jnp.float32) * (1.0 / C)

    ext_ref[:, 0:ep] = jnp.zeros((2 * NB, ep), jnp.float32)
    ext_ref[:, ep + HW:] = jnp.zeros((2 * NB, ep), jnp.float32)
    ext_ref[0:NB, ep:ep + HW] = mx
    ext_ref[NB:2 * NB, ep:ep + HW] = mn

    col = col_ref[...]
    masks = [None if kw == _PAD else
             (col >= _PAD - kw) & (col < W + _PAD - kw)
             for kw in range(_KSIZE)]

    acc = jnp.zeros((2 * NB, HW), jnp.float32)
    for kw in range(_KSIZE):
        acc_kw = jnp.zeros((2 * NB, HW), jnp.float32)
        for kh in range(_KSIZE):
            start = ep + (kh - _PAD) * W + (kw - _PAD)
            term = ext_ref[:, start:start + HW]
            acc_kw = acc_kw + (wmat_ref[:, kh * _KSIZE + kw:
                                        kh * _KSIZE + kw + 1] * term)
        if masks[kw] is not None:
            acc_kw = jnp.where(masks[kw], acc_kw, 0.0)
        acc = acc + acc_kw

    z = (acc[0:NB] + acc[NB:2 * NB]) * bn_ref[0] + bn_ref[1]
    s_ref[...] = jax.nn.sigmoid(z)[:, None, :]


@jax.jit
def _spatial_gate(x, conv_w, bn_gamma, bn_beta, bn_mean, bn_var):
    N, C, H, W = x.shape
    HW = H * W
    ep = _ext_pad(W)
    Lext = HW + 2 * ep

    NB = 1
    for cand in (8, 4, 2, 1):
        if N % cand == 0 and 4 * cand * C * HW * 4 <= (16 << 20):
            NB = cand
            break

    bn_scale = bn_gamma / jnp.sqrt(bn_var + _BN_EPS)
    bn_bias = bn_beta - bn_mean * bn_scale
    bn_params = jnp.stack([bn_scale[0], bn_bias[0]]).astype(jnp.float32)

    w2 = conv_w.reshape(2, _KSIZE * _KSIZE).astype(jnp.float32)
    wmat = jnp.repeat(w2, NB, axis=0)

    wcol = (jnp.arange(HW, dtype=jnp.int32) % W).reshape(1, HW)

    xb = x.reshape(N, C, HW)

    s = pl.pallas_call(
        functools.partial(_gate_kernel, C=C, H=H, W=W, NB=NB),
        out_shape=jax.ShapeDtypeStruct((N, 1, HW), jnp.float32),
        grid_spec=pltpu.PrefetchScalarGridSpec(
            num_scalar_prefetch=0,
            grid=(N // NB,),
            in_specs=[
                pl.BlockSpec(memory_space=pltpu.MemorySpace.SMEM),
                pl.BlockSpec((2 * NB, _KSIZE * _KSIZE), lambda n: (0, 0)),
                pl.BlockSpec((1, HW), lambda n: (0, 0)),
                pl.BlockSpec((NB, C, HW), lambda n: (n, 0, 0)),
            ],
            out_specs=pl.BlockSpec((NB, 1, HW), lambda n: (n, 0, 0)),
            scratch_shapes=[pltpu.VMEM((2 * NB, Lext), jnp.float32)],
        ),
        compiler_params=pltpu.CompilerParams(
            dimension_semantics=("parallel",),
            vmem_limit_bytes=_VMEM_LIMIT),
    )(bn_params, wmat, wcol, xb)

    return x * s.reshape(N, 1, H, W)


def kernel(x, conv_w, bn_gamma, bn_beta, bn_mean, bn_var):
    return _spatial_gate(x, conv_w, bn_gamma, bn_beta, bn_mean, bn_var)


# speedup vs baseline: 1.1026x; 1.1026x over previous
import functools

import jax
import jax.numpy as jnp
from jax.experimental import pallas as pl
from jax.experimental.pallas import tpu as pltpu

_KSIZE = 7
_PAD = _KSIZE // 2
_BN_EPS = 1e-5
_VMEM_LIMIT = 32 << 20


def _ext_pad(W):
    return ((_PAD * W + _PAD) + 127) // 128 * 128


def _gate_kernel(bn_ref, wmat_ref, col_ref, x_ref, s_ref, ext_ref,
                 *, C, H, W, NB):
    HW = H * W
    ep = _ext_pad(W)

    x = x_ref[...]
    mx = jnp.max(x, axis=1)
    mn = jnp.sum(x, axis=1, dtype=jnp.float32) * (1.0 / C)

    ext_ref[:, 0:ep] = jnp.zeros((2 * NB, ep), jnp.float32)
    ext_ref[:, ep + HW:] = jnp.zeros((2 * NB, ep), jnp.float32)
    ext_ref[0:NB, ep:ep + HW] = mx
    ext_ref[NB:2 * NB, ep:ep + HW] = mn

    col = col_ref[...]
    masks = [None if kw == _PAD else
             (col >= _PAD - kw) & (col < W + _PAD - kw)
             for kw in range(_KSIZE)]

    acc = jnp.zeros((2 * NB, HW), jnp.float32)
    for kh in range(_KSIZE):
        for kw in range(_KSIZE):
            start = ep + (kh - _PAD) * W + (kw - _PAD)
            term = ext_ref[:, start:start + HW]
            if masks[kw] is not None:
                term = jnp.where(masks[kw], term, 0.0)
            acc = acc + wmat_ref[:, kh * _KSIZE + kw:kh * _KSIZE + kw + 1] * term

    z = (acc[0:NB] + acc[NB:2 * NB]) * bn_ref[0] + bn_ref[1]
    s_ref[...] = jax.nn.sigmoid(z)[:, None, :]


@jax.jit
def _spatial_gate(x, conv_w, bn_gamma, bn_beta, bn_mean, bn_var):
    N, C, H, W = x.shape
    HW = H * W
    ep = _ext_pad(W)
    Lext = HW + 2 * ep

    NB = 1
    for cand in (8, 4, 2, 1):
        if N % cand == 0 and 4 * cand * C * HW * 4 <= (16 << 20):
            NB = cand
            break

    bn_scale = bn_gamma / jnp.sqrt(bn_var + _BN_EPS)
    bn_bias = bn_beta - bn_mean * bn_scale
    bn_params = jnp.stack([bn_scale[0], bn_bias[0]]).astype(jnp.float32)

    w2 = conv_w.reshape(2, _KSIZE * _KSIZE).astype(jnp.float32)
    wmat = jnp.repeat(w2, NB, axis=0)

    wcol = (jnp.arange(HW, dtype=jnp.int32) % W).reshape(1, HW)

    xb = x.reshape(N, C, HW)

    s = pl.pallas_call(
        functools.partial(_gate_kernel, C=C, H=H, W=W, NB=NB),
        out_shape=jax.ShapeDtypeStruct((N, 1, HW), jnp.float32),
        grid_spec=pltpu.PrefetchScalarGridSpec(
            num_scalar_prefetch=0,
            grid=(N // NB,),
            in_specs=[
                pl.BlockSpec(memory_space=pltpu.MemorySpace.SMEM),
                pl.BlockSpec((2 * NB, _KSIZE * _KSIZE), lambda n: (0, 0)),
                pl.BlockSpec((1, HW), lambda n: (0, 0)),
                pl.BlockSpec((NB, C, HW), lambda n: (n, 0, 0)),
            ],
            out_specs=pl.BlockSpec((NB, 1, HW), lambda n: (n, 0, 0)),
            scratch_shapes=[pltpu.VMEM((2 * NB, Lext), jnp.float32)],
        ),
        compiler_params=pltpu.CompilerParams(
            dimension_semantics=("parallel",),
            vmem_limit_bytes=_VMEM_LIMIT),
    )(bn_params, wmat, wcol, xb)

    return x * s.reshape(N, 1, H, W)


def kernel(x, conv_w, bn_gamma, bn_beta, bn_mean, bn_var):
    return _spatial_gate(x, conv_w, bn_gamma, bn_beta, bn_mean, bn_var)
